# baseline (speedup 1.0000x reference)
import functools

import jax
import jax.numpy as jnp
from jax import lax
from jax.experimental import pallas as pl
from jax.experimental.pallas import tpu as pltpu
from jax.experimental.pallas import tpu_sc as plsc

D = 64
SCALE = 8.0
SUB = 128
CH = 512


@functools.lru_cache(maxsize=None)
def _make_gather(B):
    info = plsc.get_sparse_core_info()
    NC, NS, L = info.num_cores, info.num_subcores, info.num_lanes
    NW = NC * NS
    assert B % (NW * CH) == 0
    PW = B // NW
    NSUB = CH // SUB
    NIT = PW // CH
    mesh = plsc.VectorSubcoreMesh(core_axis_name="c", subcore_axis_name="s")

    @functools.partial(
        pl.kernel,
        mesh=mesh,
        compiler_params=pltpu.CompilerParams(use_tc_tiling_on_sc=False),
        out_type=jax.ShapeDtypeStruct((B, D), jnp.float32),
        scratch_types=[
            pltpu.VMEM((NSUB, SUB), jnp.int32),
            pltpu.VMEM((CH, D), jnp.float32),
            pltpu.SemaphoreType.DMA,
        ],
    )
    def k(idx_hbm, table_hbm, out_hbm, idx_v, rows_v, sem):
        wid = lax.axis_index("s") * NC + lax.axis_index("c")
        base = wid * PW
        base_row = wid * (PW // SUB)

        def body(i, carry):
            off = base + i * CH
            pltpu.sync_copy(idx_hbm.at[pl.ds(base_row + i * NSUB, NSUB)], idx_v)
            copies = []
            for j in range(NSUB):
                copies.append(
                    pltpu.async_copy(
                        table_hbm.at[idx_v.at[j]],
                        rows_v.at[pl.ds(j * SUB, SUB)],
                        sem,
                    )
                )
            for c in copies:
                c.wait()

            def scale_row(r, carry2):
                for c in range(D // L):
                    sl = pl.ds(c * L, L)
                    rows_v[r, sl] = rows_v[r, sl] * SCALE
                return carry2

            lax.fori_loop(0, CH, scale_row, 0)
            pltpu.sync_copy(rows_v, out_hbm.at[pl.ds(off, CH)])
            return carry

        lax.fori_loop(0, NIT, body, 0)

    return k


def kernel(token_ids, table):
    B = token_ids.shape[0] * token_ids.shape[1]
    idx = token_ids.reshape(B // SUB, SUB)
    out = _make_gather(B)(idx, table)
    return out.reshape(token_ids.shape + (D,))


# speedup vs baseline: 1.1192x; 1.1192x over previous
import functools

import jax
import jax.numpy as jnp
from jax import lax
from jax.experimental import pallas as pl
from jax.experimental.pallas import tpu as pltpu
from jax.experimental.pallas import tpu_sc as plsc

D = 64
SCALE = 8.0
SUB = 128
CH = 512
RU = 8


@functools.lru_cache(maxsize=None)
def _make_gather(B):
    info = plsc.get_sparse_core_info()
    NC, NS, L = info.num_cores, info.num_subcores, info.num_lanes
    NW = NC * NS
    assert B % (NW * CH) == 0
    PW = B // NW
    NSUB = CH // SUB
    NIT = PW // CH
    mesh = plsc.VectorSubcoreMesh(core_axis_name="c", subcore_axis_name="s")

    @functools.partial(
        pl.kernel,
        mesh=mesh,
        compiler_params=pltpu.CompilerParams(use_tc_tiling_on_sc=False),
        out_type=jax.ShapeDtypeStruct((B, D), jnp.float32),
        scratch_types=[
            pltpu.VMEM((2, NSUB, SUB), jnp.int32),
            pltpu.VMEM((2, CH, D), jnp.float32),
            pltpu.SemaphoreType.DMA,
            pltpu.SemaphoreType.DMA,
        ],
    )
    def k(idx_hbm, table_hbm, out_hbm, idx_v, rows_v, gsem, osem):
        wid = lax.axis_index("s") * NC + lax.axis_index("c")
        base = wid * PW
        base_row = wid * (PW // SUB)

        def fire_gather(g, b):
            pltpu.sync_copy(
                idx_hbm.at[pl.ds(base_row + g * NSUB, NSUB)], idx_v.at[b]
            )
            for j in range(NSUB):
                pltpu.async_copy(
                    table_hbm.at[idx_v.at[b].at[j]],
                    rows_v.at[b].at[pl.ds(j * SUB, SUB)],
                    gsem,
                )

        def wait_gather(b):
            for j in range(NSUB):
                pltpu.make_async_copy(
                    table_hbm.at[idx_v.at[b].at[j]],
                    rows_v.at[b].at[pl.ds(j * SUB, SUB)],
                    gsem,
                ).wait()

        fire_gather(0, 0)

        def stage(g, b):
            wait_gather(b)

            @pl.when(g > 0)
            def _():
                pltpu.make_async_copy(
                    rows_v.at[1 - b],
                    out_hbm.at[pl.ds(base + (g - 1) * CH, CH)],
                    osem,
                ).wait()

            fire_gather(lax.rem(g + 1, NIT), 1 - b)

            def scale_rows(r0, carry):
                for u in range(RU):
                    r = r0 * RU + u
                    for c in range(D // L):
                        sl = pl.ds(c * L, L)
                        rows_v[b, r, sl] = rows_v[b, r, sl] * SCALE
                return carry

            lax.fori_loop(0, CH // RU, scale_rows, 0)

            pltpu.async_copy(
                rows_v.at[b], out_hbm.at[pl.ds(base + g * CH, CH)], osem
            )

        def outer(i2, carry):
            for b in range(2):
                stage(i2 * 2 + b, b)
            return carry

        lax.fori_loop(0, NIT // 2, outer, 0)

        wait_gather(0)
        pltpu.make_async_copy(
            rows_v.at[1], out_hbm.at[pl.ds(base + (NIT - 1) * CH, CH)], osem
        ).wait()

    return k


def kernel(token_ids, table):
    B = token_ids.shape[0] * token_ids.shape[1]
    idx = token_ids.reshape(B // SUB, SUB)
    out = _make_gather(B)(idx, table)
    return out.reshape(token_ids.shape + (D,))
